# Initial kernel scaffold; baseline (speedup 1.0000x reference)
import jax
import jax.numpy as jnp
from jax.experimental import pallas as pl


def kernel(h, edge_index, W_feat, b_feat, W_pool, b_pool):
    raise NotImplementedError("write your pallas kernel here")


# speedup vs baseline: 2.7579x; 2.7579x over previous
import functools

import jax
import jax.numpy as jnp
from jax import lax
from jax.experimental import pallas as pl
from jax.experimental.pallas import tpu as pltpu
from jax.experimental.pallas import tpu_sc as plsc

N = 10000
B = 20
NPG = 500
ASSIGN = 50
DIN = 256
DOUT = 512
E = 160000

HALF = DIN // 2
SPAD = 64
SHALF = SPAD // 2

NTILES = 16
CHUNK = 128
EPT = 10240
E_PAD = EPT * NTILES
NCHUNK = EPT // CHUNK

ROWS_A = 10240
ZROWS_A = ROWS_A // NTILES

NROUND = 4
NPR = N // NROUND
ROWS_C = 50048
ZROWS_C = ROWS_C // NTILES
LIVE_C = NPR * B
TAIL_C = LIVE_C - (NTILES - 1) * ZROWS_C
GARB_C = LIVE_C

@functools.cache
def _mesh():
    return plsc.VectorSubcoreMesh(core_axis_name="c", subcore_axis_name="s",
                                  num_cores=2, num_subcores=NTILES)


def _seg_body(htab, srcp, dstp, zero_a, zero_d, ones_d,
              summ_lo, summ_hi, degt,
              idx_v, idx2_v, sidx_v, rows_v, ones_v, gsem, acc, dacc):
    cid = lax.axis_index("c")
    tid = lax.axis_index("s")
    sl = pl.ds(tid * ZROWS_A, ZROWS_A)
    pltpu.sync_copy(zero_a, acc.at[sl])

    @pl.when(cid == 0)
    def _():
        pltpu.sync_copy(zero_d, dacc.at[sl])
        pltpu.sync_copy(ones_d, ones_v)

    plsc.subcore_barrier()

    def body(k, carry):
        base = tid * EPT + k * CHUNK
        pltpu.sync_copy(srcp.at[pl.ds(base, CHUNK)], idx_v)
        pltpu.sync_copy(dstp.at[pl.ds(base, CHUNK)], sidx_v)
        for j in range(CHUNK // 16):
            js = pl.ds(j * 16, 16)
            idx2_v[js] = idx_v[js] * 2 + cid
        pltpu.async_copy(htab.at[idx2_v], rows_v, gsem).wait()
        pltpu.sync_copy(rows_v, acc.at[sidx_v], add=True)

        @pl.when(cid == 0)
        def _():
            pltpu.sync_copy(ones_v, dacc.at[sidx_v], add=True)

        return carry

    lax.fori_loop(0, NCHUNK, body, 0)
    plsc.subcore_barrier()

    @pl.when(cid == 0)
    def _():
        pltpu.sync_copy(acc.at[sl], summ_lo.at[sl])
        pltpu.sync_copy(dacc.at[sl], degt.at[sl])

    @pl.when(cid == 1)
    def _():
        pltpu.sync_copy(acc.at[sl], summ_hi.at[sl])


@functools.cache
def _seg_call():
    return pl.kernel(
        _seg_body,
        out_type=[jax.ShapeDtypeStruct((ROWS_A, HALF), jnp.float32),
                  jax.ShapeDtypeStruct((ROWS_A, HALF), jnp.float32),
                  jax.ShapeDtypeStruct((ROWS_A, 16), jnp.float32)],
        mesh=_mesh(),
        scratch_types=[
            pltpu.VMEM((CHUNK,), jnp.int32),
            pltpu.VMEM((CHUNK,), jnp.int32),
            pltpu.VMEM((CHUNK,), jnp.int32),
            pltpu.VMEM((CHUNK, HALF), jnp.float32),
            pltpu.VMEM((CHUNK, 16), jnp.float32),
            pltpu.SemaphoreType.DMA,
            pltpu.VMEM_SHARED((ROWS_A, HALF), jnp.float32),
            pltpu.VMEM_SHARED((ROWS_A, 16), jnp.float32),
        ],
        compiler_params=pltpu.CompilerParams(use_tc_tiling_on_sc=False),
    )


def _as_body(slo, shi, srcp, rowidp, rndp, zero_c,
             as0, as1,
             idx_v, row_v, rnd_v, sidx_v, rows_v, gsem, acc):
    cid = lax.axis_index("c")
    tid = lax.axis_index("s")

    for r in range(NROUND):
        pltpu.sync_copy(zero_c, acc.at[pl.ds(tid * ZROWS_C, ZROWS_C)])
        plsc.subcore_barrier()

        def body(k, carry):
            base = tid * EPT + k * CHUNK
            pltpu.sync_copy(srcp.at[pl.ds(base, CHUNK)], idx_v)
            pltpu.sync_copy(rowidp.at[pl.ds(base, CHUNK)], row_v)
            pltpu.sync_copy(rndp.at[pl.ds(base, CHUNK)], rnd_v)
            for j in range(CHUNK // 16):
                js = pl.ds(j * 16, 16)
                keep = rnd_v[js] == r
                sidx_v[js] = jnp.where(keep, row_v[js], GARB_C)

            @pl.when(cid == 0)
            def _():
                pltpu.async_copy(slo.at[idx_v], rows_v, gsem).wait()

            @pl.when(cid == 1)
            def _():
                pltpu.async_copy(shi.at[idx_v], rows_v, gsem).wait()

            pltpu.sync_copy(rows_v, acc.at[sidx_v], add=True)
            return carry

        lax.fori_loop(0, NCHUNK, body, 0)
        plsc.subcore_barrier()

        off = r * LIVE_C

        @pl.when((cid == 0) & (tid < NTILES - 1))
        def _():
            pltpu.sync_copy(acc.at[pl.ds(tid * ZROWS_C, ZROWS_C)],
                            as0.at[pl.ds(off + tid * ZROWS_C, ZROWS_C)])

        @pl.when((cid == 0) & (tid == NTILES - 1))
        def _():
            pltpu.sync_copy(acc.at[pl.ds(tid * ZROWS_C, TAIL_C)],
                            as0.at[pl.ds(off + tid * ZROWS_C, TAIL_C)])

        @pl.when((cid == 1) & (tid < NTILES - 1))
        def _():
            pltpu.sync_copy(acc.at[pl.ds(tid * ZROWS_C, ZROWS_C)],
                            as1.at[pl.ds(off + tid * ZROWS_C, ZROWS_C)])

        @pl.when((cid == 1) & (tid == NTILES - 1))
        def _():
            pltpu.sync_copy(acc.at[pl.ds(tid * ZROWS_C, TAIL_C)],
                            as1.at[pl.ds(off + tid * ZROWS_C, TAIL_C)])

        plsc.subcore_barrier()


@functools.cache
def _as_call():
    return pl.kernel(
        _as_body,
        out_type=[jax.ShapeDtypeStruct((NROUND * LIVE_C, SHALF), jnp.float32),
                  jax.ShapeDtypeStruct((NROUND * LIVE_C, SHALF), jnp.float32)],
        mesh=_mesh(),
        scratch_types=[
            pltpu.VMEM((CHUNK,), jnp.int32),
            pltpu.VMEM((CHUNK,), jnp.int32),
            pltpu.VMEM((CHUNK,), jnp.int32),
            pltpu.VMEM((CHUNK,), jnp.int32),
            pltpu.VMEM((CHUNK, SHALF), jnp.float32),
            pltpu.SemaphoreType.DMA,
            pltpu.VMEM_SHARED((ROWS_C, SHALF), jnp.float32),
        ],
        compiler_params=pltpu.CompilerParams(use_tc_tiling_on_sc=False),
    )


def _dense_body(h_ref, slo_ref, shi_ref, deg_ref, wf_ref, bf_ref,
                wp_ref, bp_ref, feat_ref, s_ref):
    hb = h_ref[...]
    inv = 1.0 / jnp.maximum(deg_ref[...][:, 0:1], 1.0)
    cb = jnp.concatenate([slo_ref[...], shi_ref[...]], axis=1) * inv
    z = (jnp.dot(hb, wf_ref[0:DIN, :], preferred_element_type=jnp.float32)
         + jnp.dot(cb, wf_ref[DIN:, :], preferred_element_type=jnp.float32)
         + bf_ref[...])
    feat_ref[...] = jnp.maximum(z, 0.0)
    a = (jnp.dot(hb, wp_ref[0:DIN, :], preferred_element_type=jnp.float32)
         + jnp.dot(cb, wp_ref[DIN:, :], preferred_element_type=jnp.float32)
         + bp_ref[...])
    a = jnp.maximum(a, 0.0)
    col = lax.broadcasted_iota(jnp.int32, a.shape, 1)
    logits = jnp.where(col < ASSIGN, a, -1e30)
    m = jnp.max(logits, axis=1, keepdims=True)
    ex = jnp.exp(logits - m)
    s_ref[...] = ex / jnp.sum(ex, axis=1, keepdims=True)


_RB = 1000

_dense_call = pl.pallas_call(
    _dense_body,
    grid=(N // _RB,),
    in_specs=[
        pl.BlockSpec((_RB, DIN), lambda i: (i, 0)),
        pl.BlockSpec((_RB, HALF), lambda i: (i, 0)),
        pl.BlockSpec((_RB, HALF), lambda i: (i, 0)),
        pl.BlockSpec((_RB, 16), lambda i: (i, 0)),
        pl.BlockSpec((2 * DIN, DOUT), lambda i: (0, 0)),
        pl.BlockSpec((1, DOUT), lambda i: (0, 0)),
        pl.BlockSpec((2 * DIN, SPAD), lambda i: (0, 0)),
        pl.BlockSpec((1, SPAD), lambda i: (0, 0)),
    ],
    out_specs=[
        pl.BlockSpec((_RB, DOUT), lambda i: (i, 0)),
        pl.BlockSpec((_RB, SPAD), lambda i: (i, 0)),
    ],
    out_shape=[jax.ShapeDtypeStruct((N, DOUT), jnp.float32),
               jax.ShapeDtypeStruct((N, SPAD), jnp.float32)],
)


def _pool_body(s_ref, feat_ref, as0_ref, as1_ref,
               hnew_ref, adj0_ref, adj1_ref):
    s = s_ref[...][0]
    dn = (((0,), (0,)), ((), ()))
    hnew_ref[...] = lax.dot_general(s, feat_ref[...][0], dn,
                                    preferred_element_type=jnp.float32)[None]
    adj0_ref[...] = lax.dot_general(s, as0_ref[...][0], dn,
                                    preferred_element_type=jnp.float32)[None]
    adj1_ref[...] = lax.dot_general(s, as1_ref[...][0], dn,
                                    preferred_element_type=jnp.float32)[None]


_AW = B * SHALF

_pool_call = pl.pallas_call(
    _pool_body,
    grid=(B,),
    in_specs=[
        pl.BlockSpec((1, NPG, SPAD), lambda g: (g, 0, 0)),
        pl.BlockSpec((1, NPG, DOUT), lambda g: (g, 0, 0)),
        pl.BlockSpec((1, NPG, _AW), lambda g: (g, 0, 0)),
        pl.BlockSpec((1, NPG, _AW), lambda g: (g, 0, 0)),
    ],
    out_specs=[
        pl.BlockSpec((1, SPAD, DOUT), lambda g: (g, 0, 0)),
        pl.BlockSpec((1, SPAD, _AW), lambda g: (g, 0, 0)),
        pl.BlockSpec((1, SPAD, _AW), lambda g: (g, 0, 0)),
    ],
    out_shape=[jax.ShapeDtypeStruct((B, SPAD, DOUT), jnp.float32),
               jax.ShapeDtypeStruct((B, SPAD, _AW), jnp.float32),
               jax.ShapeDtypeStruct((B, SPAD, _AW), jnp.float32)],
)


@jax.jit
def kernel(h, edge_index, W_feat, b_feat, W_pool, b_pool):
    src = edge_index[0]
    dst = edge_index[1]
    pad = E_PAD - E
    srcp = jnp.concatenate([src, jnp.zeros((pad,), jnp.int32)])
    dstp = jnp.concatenate([dst, jnp.full((pad,), N, jnp.int32)])
    rowidp = (dstp % NPR) * B + srcp // NPG
    rndp = dstp // NPR

    htab = h.reshape(2 * N, HALF)
    zero_a = jnp.zeros((ZROWS_A, HALF), jnp.float32)
    zero_d = jnp.zeros((ZROWS_A, 16), jnp.float32)
    ones_d = jnp.ones((CHUNK, 16), jnp.float32)
    summ_lo, summ_hi, degt = _seg_call()(htab, srcp, dstp,
                                         zero_a, zero_d, ones_d)

    feat, s_pad = _dense_call(h, summ_lo[:N], summ_hi[:N], degt[:N],
                              W_feat, b_feat.reshape(1, DOUT),
                              jnp.pad(W_pool, ((0, 0), (0, SPAD - ASSIGN))),
                              jnp.pad(b_pool, (0, SPAD - ASSIGN)).reshape(1, SPAD))

    slo = s_pad[:, :SHALF]
    shi = s_pad[:, SHALF:]
    zero_c = jnp.zeros((ZROWS_C, SHALF), jnp.float32)
    as0, as1 = _as_call()(slo, shi, srcp, rowidp, rndp, zero_c)

    hnew, adj0, adj1 = _pool_call(s_pad.reshape(B, NPG, SPAD),
                                  feat.reshape(B, NPG, DOUT),
                                  as0.reshape(B, NPG, _AW),
                                  as1.reshape(B, NPG, _AW))

    h_new = hnew[:, :ASSIGN, :].reshape(B * ASSIGN, DOUT)
    a0 = adj0.reshape(B, SPAD, B, SHALF)
    a1 = adj1.reshape(B, SPAD, B, SHALF)
    adj = jnp.concatenate([a0, a1], axis=-1)[:, :ASSIGN, :, :ASSIGN]
    adj_new = adj.reshape(B * ASSIGN, B * ASSIGN)
    return (adj_new, h_new)


# speedup vs baseline: 3.7583x; 1.3627x over previous
import functools

import jax
import jax.numpy as jnp
from jax import lax
from jax.experimental import pallas as pl
from jax.experimental.pallas import tpu as pltpu
from jax.experimental.pallas import tpu_sc as plsc

N = 10000
B = 20
NPG = 500
ASSIGN = 50
DIN = 256
DOUT = 512
E = 160000

HALF = DIN // 2
SPAD = 64
SHALF = SPAD // 2

NTILES = 16
CHUNK = 128
EPT = 10240
E_PAD = EPT * NTILES
NCHUNK = EPT // CHUNK
NSTEP = NCHUNK // 2

ROWS_A = 10240
ZROWS_A = ROWS_A // NTILES

NROUND = 4
NPR = N // NROUND
ROWS_C = 50048
ZROWS_C = ROWS_C // NTILES
LIVE_C = NPR * B
TAIL_C = LIVE_C - (NTILES - 1) * ZROWS_C
GARB_C = LIVE_C

@functools.cache
def _mesh():
    return plsc.VectorSubcoreMesh(core_axis_name="c", subcore_axis_name="s",
                                  num_cores=2, num_subcores=NTILES)


def _seg_body(htab, epack, zero_a, zero_d, ones_d,
              summ_lo, summ_hi, degt,
              ebuf0, ebuf1, idx0, idx1, sidx0, sidx1, rows0, rows1, ones_v,
              gsem0, gsem1, ssem0, ssem1, dsem0, dsem1, acc, dacc):
    cid = lax.axis_index("c")
    tid = lax.axis_index("s")
    sl = pl.ds(tid * ZROWS_A, ZROWS_A)
    pltpu.sync_copy(zero_a, acc.at[sl])

    @pl.when(cid == 0)
    def _():
        pltpu.sync_copy(zero_d, dacc.at[sl])
        pltpu.sync_copy(ones_d, ones_v)

    plsc.subcore_barrier()
    base = tid * NCHUNK

    def compute_idx(ebuf, idxr, sidxr):
        for j in range(CHUNK // 16):
            js = pl.ds(j * 16, 16)
            idxr[js] = ebuf[0, js] * 2 + cid
            sidxr[js] = ebuf[1, js]

    def issue_gather(idxr, rows, gsem):
        pltpu.async_copy(htab.at[idxr], rows, gsem)

    def wait_gather(idxr, rows, gsem):
        pltpu.make_async_copy(htab.at[idxr], rows, gsem).wait()

    def issue_scatter(rows, sidxr, ssem, dsem):
        pltpu.async_copy(rows, acc.at[sidxr], ssem, add=True)

        @pl.when(cid == 0)
        def _():
            pltpu.async_copy(ones_v, dacc.at[sidxr], dsem, add=True)

    def wait_scatter(rows, sidxr, ssem, dsem):
        pltpu.make_async_copy(rows, acc.at[sidxr], ssem).wait()

        @pl.when(cid == 0)
        def _():
            pltpu.make_async_copy(ones_v, dacc.at[sidxr], dsem).wait()

    pltpu.sync_copy(epack.at[base], ebuf0)
    compute_idx(ebuf0, idx0, sidx0)
    issue_gather(idx0, rows0, gsem0)

    def step(i, carry):
        pltpu.sync_copy(epack.at[base + 2 * i + 1], ebuf1)

        @pl.when(i > 0)
        def _():
            wait_scatter(rows1, sidx1, ssem1, dsem1)

        compute_idx(ebuf1, idx1, sidx1)
        issue_gather(idx1, rows1, gsem1)
        wait_gather(idx0, rows0, gsem0)
        issue_scatter(rows0, sidx0, ssem0, dsem0)

        @pl.when(i < NSTEP - 1)
        def _():
            pltpu.sync_copy(epack.at[base + 2 * i + 2], ebuf0)
            wait_scatter(rows0, sidx0, ssem0, dsem0)
            compute_idx(ebuf0, idx0, sidx0)
            issue_gather(idx0, rows0, gsem0)

        wait_gather(idx1, rows1, gsem1)
        issue_scatter(rows1, sidx1, ssem1, dsem1)
        return carry

    lax.fori_loop(0, NSTEP, step, 0)
    wait_scatter(rows0, sidx0, ssem0, dsem0)
    wait_scatter(rows1, sidx1, ssem1, dsem1)
    plsc.subcore_barrier()

    @pl.when(cid == 0)
    def _():
        pltpu.sync_copy(acc.at[sl], summ_lo.at[sl])
        pltpu.sync_copy(dacc.at[sl], degt.at[sl])

    @pl.when(cid == 1)
    def _():
        pltpu.sync_copy(acc.at[sl], summ_hi.at[sl])


@functools.cache
def _seg_call():
    return pl.kernel(
        _seg_body,
        out_type=[jax.ShapeDtypeStruct((ROWS_A, HALF), jnp.float32),
                  jax.ShapeDtypeStruct((ROWS_A, HALF), jnp.float32),
                  jax.ShapeDtypeStruct((ROWS_A, 16), jnp.float32)],
        mesh=_mesh(),
        scratch_types=[
            pltpu.VMEM((2, CHUNK), jnp.int32),
            pltpu.VMEM((2, CHUNK), jnp.int32),
            pltpu.VMEM((CHUNK,), jnp.int32),
            pltpu.VMEM((CHUNK,), jnp.int32),
            pltpu.VMEM((CHUNK,), jnp.int32),
            pltpu.VMEM((CHUNK,), jnp.int32),
            pltpu.VMEM((CHUNK, HALF), jnp.float32),
            pltpu.VMEM((CHUNK, HALF), jnp.float32),
            pltpu.VMEM((CHUNK, 16), jnp.float32),
            pltpu.SemaphoreType.DMA,
            pltpu.SemaphoreType.DMA,
            pltpu.SemaphoreType.DMA,
            pltpu.SemaphoreType.DMA,
            pltpu.SemaphoreType.DMA,
            pltpu.SemaphoreType.DMA,
            pltpu.VMEM_SHARED((ROWS_A, HALF), jnp.float32),
            pltpu.VMEM_SHARED((ROWS_A, 16), jnp.float32),
        ],
        compiler_params=pltpu.CompilerParams(use_tc_tiling_on_sc=False),
    )


def _as_body(slo, shi, epack, zero_c,
             as0, as1,
             ebuf0, ebuf1, sidx0, sidx1, rows0, rows1,
             gsem0, gsem1, ssem0, ssem1, acc):
    cid = lax.axis_index("c")
    tid = lax.axis_index("s")
    base = tid * NCHUNK

    def issue_gather(ebuf, rows, gsem):
        @pl.when(cid == 0)
        def _():
            pltpu.async_copy(slo.at[ebuf.at[0]], rows, gsem)

        @pl.when(cid == 1)
        def _():
            pltpu.async_copy(shi.at[ebuf.at[0]], rows, gsem)

    def wait_gather(ebuf, rows, gsem):
        pltpu.make_async_copy(slo.at[ebuf.at[0]], rows, gsem).wait()

    for r in range(NROUND):
        pltpu.sync_copy(zero_c, acc.at[pl.ds(tid * ZROWS_C, ZROWS_C)])
        plsc.subcore_barrier()

        def compute_sidx(ebuf, sidxr):
            for j in range(CHUNK // 16):
                js = pl.ds(j * 16, 16)
                keep = ebuf[2, js] == r
                sidxr[js] = jnp.where(keep, ebuf[1, js], GARB_C)

        pltpu.sync_copy(epack.at[base], ebuf0)
        compute_sidx(ebuf0, sidx0)
        issue_gather(ebuf0, rows0, gsem0)

        def step(i, carry):
            pltpu.sync_copy(epack.at[base + 2 * i + 1], ebuf1)

            @pl.when(i > 0)
            def _():
                pltpu.make_async_copy(rows1, acc.at[sidx1], ssem1).wait()

            compute_sidx(ebuf1, sidx1)
            issue_gather(ebuf1, rows1, gsem1)
            wait_gather(ebuf0, rows0, gsem0)
            pltpu.async_copy(rows0, acc.at[sidx0], ssem0, add=True)

            @pl.when(i < NSTEP - 1)
            def _():
                pltpu.sync_copy(epack.at[base + 2 * i + 2], ebuf0)
                pltpu.make_async_copy(rows0, acc.at[sidx0], ssem0).wait()
                compute_sidx(ebuf0, sidx0)
                issue_gather(ebuf0, rows0, gsem0)

            wait_gather(ebuf1, rows1, gsem1)
            pltpu.async_copy(rows1, acc.at[sidx1], ssem1, add=True)
            return carry

        lax.fori_loop(0, NSTEP, step, 0)
        pltpu.make_async_copy(rows0, acc.at[sidx0], ssem0).wait()
        pltpu.make_async_copy(rows1, acc.at[sidx1], ssem1).wait()
        plsc.subcore_barrier()

        off = r * LIVE_C

        @pl.when((cid == 0) & (tid < NTILES - 1))
        def _():
            pltpu.sync_copy(acc.at[pl.ds(tid * ZROWS_C, ZROWS_C)],
                            as0.at[pl.ds(off + tid * ZROWS_C, ZROWS_C)])

        @pl.when((cid == 0) & (tid == NTILES - 1))
        def _():
            pltpu.sync_copy(acc.at[pl.ds(tid * ZROWS_C, TAIL_C)],
                            as0.at[pl.ds(off + tid * ZROWS_C, TAIL_C)])

        @pl.when((cid == 1) & (tid < NTILES - 1))
        def _():
            pltpu.sync_copy(acc.at[pl.ds(tid * ZROWS_C, ZROWS_C)],
                            as1.at[pl.ds(off + tid * ZROWS_C, ZROWS_C)])

        @pl.when((cid == 1) & (tid == NTILES - 1))
        def _():
            pltpu.sync_copy(acc.at[pl.ds(tid * ZROWS_C, TAIL_C)],
                            as1.at[pl.ds(off + tid * ZROWS_C, TAIL_C)])

        plsc.subcore_barrier()


@functools.cache
def _as_call():
    return pl.kernel(
        _as_body,
        out_type=[jax.ShapeDtypeStruct((NROUND * LIVE_C, SHALF), jnp.float32),
                  jax.ShapeDtypeStruct((NROUND * LIVE_C, SHALF), jnp.float32)],
        mesh=_mesh(),
        scratch_types=[
            pltpu.VMEM((3, CHUNK), jnp.int32),
            pltpu.VMEM((3, CHUNK), jnp.int32),
            pltpu.VMEM((CHUNK,), jnp.int32),
            pltpu.VMEM((CHUNK,), jnp.int32),
            pltpu.VMEM((CHUNK, SHALF), jnp.float32),
            pltpu.VMEM((CHUNK, SHALF), jnp.float32),
            pltpu.SemaphoreType.DMA,
            pltpu.SemaphoreType.DMA,
            pltpu.SemaphoreType.DMA,
            pltpu.SemaphoreType.DMA,
            pltpu.VMEM_SHARED((ROWS_C, SHALF), jnp.float32),
        ],
        compiler_params=pltpu.CompilerParams(use_tc_tiling_on_sc=False),
    )


def _dense_body(h_ref, slo_ref, shi_ref, deg_ref, wf_ref, bf_ref,
                wp_ref, bp_ref, feat_ref, s_ref):
    hb = h_ref[...]
    inv = 1.0 / jnp.maximum(deg_ref[...][:, 0:1], 1.0)
    cb = jnp.concatenate([slo_ref[...], shi_ref[...]], axis=1) * inv
    z = (jnp.dot(hb, wf_ref[0:DIN, :], preferred_element_type=jnp.float32)
         + jnp.dot(cb, wf_ref[DIN:, :], preferred_element_type=jnp.float32)
         + bf_ref[...])
    feat_ref[...] = jnp.maximum(z, 0.0)
    a = (jnp.dot(hb, wp_ref[0:DIN, :], preferred_element_type=jnp.float32)
         + jnp.dot(cb, wp_ref[DIN:, :], preferred_element_type=jnp.float32)
         + bp_ref[...])
    a = jnp.maximum(a, 0.0)
    col = lax.broadcasted_iota(jnp.int32, a.shape, 1)
    logits = jnp.where(col < ASSIGN, a, -1e30)
    m = jnp.max(logits, axis=1, keepdims=True)
    ex = jnp.exp(logits - m)
    s_ref[...] = ex / jnp.sum(ex, axis=1, keepdims=True)


_RB = 1000

_dense_call = pl.pallas_call(
    _dense_body,
    grid=(N // _RB,),
    in_specs=[
        pl.BlockSpec((_RB, DIN), lambda i: (i, 0)),
        pl.BlockSpec((_RB, HALF), lambda i: (i, 0)),
        pl.BlockSpec((_RB, HALF), lambda i: (i, 0)),
        pl.BlockSpec((_RB, 16), lambda i: (i, 0)),
        pl.BlockSpec((2 * DIN, DOUT), lambda i: (0, 0)),
        pl.BlockSpec((1, DOUT), lambda i: (0, 0)),
        pl.BlockSpec((2 * DIN, SPAD), lambda i: (0, 0)),
        pl.BlockSpec((1, SPAD), lambda i: (0, 0)),
    ],
    out_specs=[
        pl.BlockSpec((_RB, DOUT), lambda i: (i, 0)),
        pl.BlockSpec((_RB, SPAD), lambda i: (i, 0)),
    ],
    out_shape=[jax.ShapeDtypeStruct((N, DOUT), jnp.float32),
               jax.ShapeDtypeStruct((N, SPAD), jnp.float32)],
)


def _pool_body(s_ref, feat_ref, as0_ref, as1_ref,
               hnew_ref, adj0_ref, adj1_ref):
    s = s_ref[...][0]
    dn = (((0,), (0,)), ((), ()))
    hnew_ref[...] = lax.dot_general(s, feat_ref[...][0], dn,
                                    preferred_element_type=jnp.float32)[None]
    adj0_ref[...] = lax.dot_general(s, as0_ref[...][0], dn,
                                    preferred_element_type=jnp.float32)[None]
    adj1_ref[...] = lax.dot_general(s, as1_ref[...][0], dn,
                                    preferred_element_type=jnp.float32)[None]


_AW = B * SHALF

_pool_call = pl.pallas_call(
    _pool_body,
    grid=(B,),
    in_specs=[
        pl.BlockSpec((1, NPG, SPAD), lambda g: (g, 0, 0)),
        pl.BlockSpec((1, NPG, DOUT), lambda g: (g, 0, 0)),
        pl.BlockSpec((1, NPG, _AW), lambda g: (g, 0, 0)),
        pl.BlockSpec((1, NPG, _AW), lambda g: (g, 0, 0)),
    ],
    out_specs=[
        pl.BlockSpec((1, SPAD, DOUT), lambda g: (g, 0, 0)),
        pl.BlockSpec((1, SPAD, _AW), lambda g: (g, 0, 0)),
        pl.BlockSpec((1, SPAD, _AW), lambda g: (g, 0, 0)),
    ],
    out_shape=[jax.ShapeDtypeStruct((B, SPAD, DOUT), jnp.float32),
               jax.ShapeDtypeStruct((B, SPAD, _AW), jnp.float32),
               jax.ShapeDtypeStruct((B, SPAD, _AW), jnp.float32)],
)


@jax.jit
def kernel(h, edge_index, W_feat, b_feat, W_pool, b_pool):
    src = edge_index[0]
    dst = edge_index[1]
    pad = E_PAD - E
    srcp = jnp.concatenate([src, jnp.zeros((pad,), jnp.int32)])
    dstp = jnp.concatenate([dst, jnp.full((pad,), N, jnp.int32)])
    rowidp = (dstp % NPR) * B + srcp // NPG
    rndp = dstp // NPR

    epack_a = jnp.stack([srcp.reshape(-1, CHUNK), dstp.reshape(-1, CHUNK)],
                        axis=1)
    epack_c = jnp.stack([srcp.reshape(-1, CHUNK), rowidp.reshape(-1, CHUNK),
                         rndp.reshape(-1, CHUNK)], axis=1)

    htab = h.reshape(2 * N, HALF)
    zero_a = jnp.zeros((ZROWS_A, HALF), jnp.float32)
    zero_d = jnp.zeros((ZROWS_A, 16), jnp.float32)
    ones_d = jnp.ones((CHUNK, 16), jnp.float32)
    summ_lo, summ_hi, degt = _seg_call()(htab, epack_a,
                                         zero_a, zero_d, ones_d)

    feat, s_pad = _dense_call(h, summ_lo[:N], summ_hi[:N], degt[:N],
                              W_feat, b_feat.reshape(1, DOUT),
                              jnp.pad(W_pool, ((0, 0), (0, SPAD - ASSIGN))),
                              jnp.pad(b_pool, (0, SPAD - ASSIGN)).reshape(1, SPAD))

    slo = s_pad[:, :SHALF]
    shi = s_pad[:, SHALF:]
    zero_c = jnp.zeros((ZROWS_C, SHALF), jnp.float32)
    as0, as1 = _as_call()(slo, shi, epack_c, zero_c)

    hnew, adj0, adj1 = _pool_call(s_pad.reshape(B, NPG, SPAD),
                                  feat.reshape(B, NPG, DOUT),
                                  as0.reshape(B, NPG, _AW),
                                  as1.reshape(B, NPG, _AW))

    h_new = hnew[:, :ASSIGN, :].reshape(B * ASSIGN, DOUT)
    a0 = adj0.reshape(B, SPAD, B, SHALF)
    a1 = adj1.reshape(B, SPAD, B, SHALF)
    adj = jnp.concatenate([a0, a1], axis=-1)[:, :ASSIGN, :, :ASSIGN]
    adj_new = adj.reshape(B * ASSIGN, B * ASSIGN)
    return (adj_new, h_new)


# speedup vs baseline: 3.7994x; 1.0109x over previous
import functools

import jax
import jax.numpy as jnp
from jax import lax
from jax.experimental import pallas as pl
from jax.experimental.pallas import tpu as pltpu
from jax.experimental.pallas import tpu_sc as plsc

N = 10000
B = 20
NPG = 500
ASSIGN = 50
DIN = 256
DOUT = 512
E = 160000

HALF = DIN // 2
SPAD = 64
SHALF = SPAD // 2

NTILES = 16
CHUNK = 128
EPT = 10240
E_PAD = EPT * NTILES
NCHUNK = EPT // CHUNK
NBUF = 4
NSTEP4 = NCHUNK // NBUF
CHUNK_A = 64
NCHUNK_A = EPT // CHUNK_A
NSTEP_A = NCHUNK_A // NBUF

ROWS_A = 10240
ZROWS_A = ROWS_A // NTILES

NROUND = 4
NPR = N // NROUND
ROWS_C = 50048
ZROWS_C = ROWS_C // NTILES
LIVE_C = NPR * B
TAIL_C = LIVE_C - (NTILES - 1) * ZROWS_C
GARB_C = LIVE_C

@functools.cache
def _mesh():
    return plsc.VectorSubcoreMesh(core_axis_name="c", subcore_axis_name="s",
                                  num_cores=2, num_subcores=NTILES)


def _seg_body(htab, epack, zero_a, zero_d, ones_d,
              summ_lo, summ_hi, degt,
              e0, e1, e2, e3, i0, i1, i2, i3, s0, s1, s2, s3,
              r0, r1, r2, r3, ones_v,
              es0, es1, es2, es3, g0, g1, g2, g3,
              ss0, ss1, ss2, ss3, d0, d1, d2, d3, acc, dacc):
    cid = lax.axis_index("c")
    tid = lax.axis_index("s")
    sl = pl.ds(tid * ZROWS_A, ZROWS_A)
    pltpu.sync_copy(zero_a, acc.at[sl])

    @pl.when(cid == 0)
    def _():
        pltpu.sync_copy(zero_d, dacc.at[sl])
        pltpu.sync_copy(ones_d, ones_v)

    plsc.subcore_barrier()
    base = tid * NCHUNK_A

    ebufs = (e0, e1, e2, e3)
    idxs = (i0, i1, i2, i3)
    sidxs = (s0, s1, s2, s3)
    rowss = (r0, r1, r2, r3)
    esems = (es0, es1, es2, es3)
    gsems = (g0, g1, g2, g3)
    ssems = (ss0, ss1, ss2, ss3)
    dsems = (d0, d1, d2, d3)

    def compute_idx(j):
        for q in range(CHUNK_A // 16):
            qs = pl.ds(q * 16, 16)
            idxs[j][qs] = ebufs[j][0, qs] * 2 + cid
            sidxs[j][qs] = ebufs[j][1, qs]

    def issue_ebuf(k, j):
        pltpu.async_copy(epack.at[base + k], ebufs[j], esems[j])

    def wait_ebuf(j):
        pltpu.make_async_copy(epack.at[base], ebufs[j], esems[j]).wait()

    def issue_gather(j):
        pltpu.async_copy(htab.at[idxs[j]], rowss[j], gsems[j])

    def wait_gather(j):
        pltpu.make_async_copy(htab.at[idxs[j]], rowss[j], gsems[j]).wait()

    def issue_scatter(j):
        pltpu.async_copy(rowss[j], acc.at[sidxs[j]], ssems[j], add=True)

        @pl.when(cid == 0)
        def _():
            pltpu.async_copy(ones_v, dacc.at[sidxs[j]], dsems[j], add=True)

    def wait_scatter(j):
        pltpu.make_async_copy(rowss[j], acc.at[sidxs[j]], ssems[j]).wait()

        @pl.when(cid == 0)
        def _():
            pltpu.make_async_copy(ones_v, dacc.at[sidxs[j]], dsems[j]).wait()

    for j in (0, 1):
        pltpu.sync_copy(epack.at[base + j], ebufs[j])
        compute_idx(j)
        issue_gather(j)
    for j in (2, 3):
        issue_ebuf(j, j)

    def step(i, carry):
        for jj in range(NBUF):
            m = NBUF * i + jj
            jn = (jj + 2) % NBUF
            wait_gather(jj)
            issue_scatter(jj)

            @pl.when(i < NSTEP_A - 1)
            def _(jj=jj, m=m):
                issue_ebuf(m + 4, jj)

            if jj < 2:
                @pl.when(i > 0)
                def _(jn=jn):
                    wait_scatter(jn)

                wait_ebuf(jn)
                compute_idx(jn)
                issue_gather(jn)
            else:
                @pl.when(i < NSTEP_A - 1)
                def _(jn=jn, m=m):
                    wait_scatter(jn)
                    wait_ebuf(jn)
                    compute_idx(jn)
                    issue_gather(jn)

        return carry

    lax.fori_loop(0, NSTEP_A, step, 0)
    for j in range(NBUF):
        wait_scatter(j)
    plsc.subcore_barrier()

    @pl.when(cid == 0)
    def _():
        pltpu.sync_copy(acc.at[sl], summ_lo.at[sl])
        pltpu.sync_copy(dacc.at[sl], degt.at[sl])

    @pl.when(cid == 1)
    def _():
        pltpu.sync_copy(acc.at[sl], summ_hi.at[sl])


@functools.cache
def _seg_call():
    return pl.kernel(
        _seg_body,
        out_type=[jax.ShapeDtypeStruct((ROWS_A, HALF), jnp.float32),
                  jax.ShapeDtypeStruct((ROWS_A, HALF), jnp.float32),
                  jax.ShapeDtypeStruct((ROWS_A, 16), jnp.float32)],
        mesh=_mesh(),
        scratch_types=(
            [pltpu.VMEM((2, CHUNK_A), jnp.int32)] * 4
            + [pltpu.VMEM((CHUNK_A,), jnp.int32)] * 8
            + [pltpu.VMEM((CHUNK_A, HALF), jnp.float32)] * 4
            + [pltpu.VMEM((CHUNK_A, 16), jnp.float32)]
            + [pltpu.SemaphoreType.DMA] * 16
            + [pltpu.VMEM_SHARED((ROWS_A, HALF), jnp.float32),
               pltpu.VMEM_SHARED((ROWS_A, 16), jnp.float32)]
        ),
        compiler_params=pltpu.CompilerParams(use_tc_tiling_on_sc=False),
    )


def _as_body(slo, shi, epack, zero_c,
             as0, as1,
             e0, e1, e2, e3, s0, s1, s2, s3, r0, r1, r2, r3,
             es0, es1, es2, es3, g0, g1, g2, g3, ss0, ss1, ss2, ss3, acc):
    cid = lax.axis_index("c")
    tid = lax.axis_index("s")
    base = tid * NCHUNK
    ebufs = (e0, e1, e2, e3)
    sidxs = (s0, s1, s2, s3)
    rowss = (r0, r1, r2, r3)
    esems = (es0, es1, es2, es3)
    gsems = (g0, g1, g2, g3)
    ssems = (ss0, ss1, ss2, ss3)

    def issue_ebuf(k, j):
        pltpu.async_copy(epack.at[base + k], ebufs[j], esems[j])

    def wait_ebuf(j):
        pltpu.make_async_copy(epack.at[base], ebufs[j], esems[j]).wait()

    def issue_gather(j):
        @pl.when(cid == 0)
        def _():
            pltpu.async_copy(slo.at[ebufs[j].at[0]], rowss[j], gsems[j])

        @pl.when(cid == 1)
        def _():
            pltpu.async_copy(shi.at[ebufs[j].at[0]], rowss[j], gsems[j])

    def wait_gather(j):
        pltpu.make_async_copy(slo.at[ebufs[j].at[0]], rowss[j],
                              gsems[j]).wait()

    def issue_scatter(j):
        pltpu.async_copy(rowss[j], acc.at[sidxs[j]], ssems[j], add=True)

    def wait_scatter(j):
        pltpu.make_async_copy(rowss[j], acc.at[sidxs[j]], ssems[j]).wait()

    for r in range(NROUND):
        pltpu.sync_copy(zero_c, acc.at[pl.ds(tid * ZROWS_C, ZROWS_C)])
        plsc.subcore_barrier()

        def compute_sidx(j):
            for q in range(CHUNK // 16):
                qs = pl.ds(q * 16, 16)
                keep = ebufs[j][2, qs] == r
                sidxs[j][qs] = jnp.where(keep, ebufs[j][1, qs], GARB_C)

        for j in (0, 1):
            pltpu.sync_copy(epack.at[base + j], ebufs[j])
            compute_sidx(j)
            issue_gather(j)
        for j in (2, 3):
            issue_ebuf(j, j)

        def step(i, carry):
            for jj in range(NBUF):
                m = NBUF * i + jj
                jn = (jj + 2) % NBUF
                wait_gather(jj)
                issue_scatter(jj)

                @pl.when(i < NSTEP4 - 1)
                def _(jj=jj, m=m):
                    issue_ebuf(m + 4, jj)

                if jj < 2:
                    @pl.when(i > 0)
                    def _(jn=jn):
                        wait_scatter(jn)

                    wait_ebuf(jn)
                    compute_sidx(jn)
                    issue_gather(jn)
                else:
                    @pl.when(i < NSTEP4 - 1)
                    def _(jn=jn, m=m):
                        wait_scatter(jn)
                        wait_ebuf(jn)
                        compute_sidx(jn)
                        issue_gather(jn)

            return carry

        lax.fori_loop(0, NSTEP4, step, 0)
        for j in range(NBUF):
            wait_scatter(j)
        plsc.subcore_barrier()

        off = r * LIVE_C

        @pl.when((cid == 0) & (tid < NTILES - 1))
        def _():
            pltpu.sync_copy(acc.at[pl.ds(tid * ZROWS_C, ZROWS_C)],
                            as0.at[pl.ds(off + tid * ZROWS_C, ZROWS_C)])

        @pl.when((cid == 0) & (tid == NTILES - 1))
        def _():
            pltpu.sync_copy(acc.at[pl.ds(tid * ZROWS_C, TAIL_C)],
                            as0.at[pl.ds(off + tid * ZROWS_C, TAIL_C)])

        @pl.when((cid == 1) & (tid < NTILES - 1))
        def _():
            pltpu.sync_copy(acc.at[pl.ds(tid * ZROWS_C, ZROWS_C)],
                            as1.at[pl.ds(off + tid * ZROWS_C, ZROWS_C)])

        @pl.when((cid == 1) & (tid == NTILES - 1))
        def _():
            pltpu.sync_copy(acc.at[pl.ds(tid * ZROWS_C, TAIL_C)],
                            as1.at[pl.ds(off + tid * ZROWS_C, TAIL_C)])

        plsc.subcore_barrier()


@functools.cache
def _as_call():
    return pl.kernel(
        _as_body,
        out_type=[jax.ShapeDtypeStruct((NROUND * LIVE_C, SHALF), jnp.float32),
                  jax.ShapeDtypeStruct((NROUND * LIVE_C, SHALF), jnp.float32)],
        mesh=_mesh(),
        scratch_types=(
            [pltpu.VMEM((3, CHUNK), jnp.int32)] * 4
            + [pltpu.VMEM((CHUNK,), jnp.int32)] * 4
            + [pltpu.VMEM((CHUNK, SHALF), jnp.float32)] * 4
            + [pltpu.SemaphoreType.DMA] * 12
            + [pltpu.VMEM_SHARED((ROWS_C, SHALF), jnp.float32)]
        ),
        compiler_params=pltpu.CompilerParams(use_tc_tiling_on_sc=False),
    )


def _dense_body(h_ref, slo_ref, shi_ref, deg_ref, wf_ref, bf_ref,
                wp_ref, bp_ref, feat_ref, s_ref):
    hb = h_ref[...]
    inv = 1.0 / jnp.maximum(deg_ref[...][:, 0:1], 1.0)
    cb = jnp.concatenate([slo_ref[...], shi_ref[...]], axis=1) * inv
    z = (jnp.dot(hb, wf_ref[0:DIN, :], preferred_element_type=jnp.float32)
         + jnp.dot(cb, wf_ref[DIN:, :], preferred_element_type=jnp.float32)
         + bf_ref[...])
    feat_ref[...] = jnp.maximum(z, 0.0)
    a = (jnp.dot(hb, wp_ref[0:DIN, :], preferred_element_type=jnp.float32)
         + jnp.dot(cb, wp_ref[DIN:, :], preferred_element_type=jnp.float32)
         + bp_ref[...])
    a = jnp.maximum(a, 0.0)
    col = lax.broadcasted_iota(jnp.int32, a.shape, 1)
    logits = jnp.where(col < ASSIGN, a, -1e30)
    m = jnp.max(logits, axis=1, keepdims=True)
    ex = jnp.exp(logits - m)
    s_ref[...] = ex / jnp.sum(ex, axis=1, keepdims=True)


_RB = 1000

_dense_call = pl.pallas_call(
    _dense_body,
    grid=(N // _RB,),
    in_specs=[
        pl.BlockSpec((_RB, DIN), lambda i: (i, 0)),
        pl.BlockSpec((_RB, HALF), lambda i: (i, 0)),
        pl.BlockSpec((_RB, HALF), lambda i: (i, 0)),
        pl.BlockSpec((_RB, 16), lambda i: (i, 0)),
        pl.BlockSpec((2 * DIN, DOUT), lambda i: (0, 0)),
        pl.BlockSpec((1, DOUT), lambda i: (0, 0)),
        pl.BlockSpec((2 * DIN, SPAD), lambda i: (0, 0)),
        pl.BlockSpec((1, SPAD), lambda i: (0, 0)),
    ],
    out_specs=[
        pl.BlockSpec((_RB, DOUT), lambda i: (i, 0)),
        pl.BlockSpec((_RB, SPAD), lambda i: (i, 0)),
    ],
    out_shape=[jax.ShapeDtypeStruct((N, DOUT), jnp.float32),
               jax.ShapeDtypeStruct((N, SPAD), jnp.float32)],
)


def _pool_body(s_ref, feat_ref, as0_ref, as1_ref,
               hnew_ref, adj0_ref, adj1_ref):
    s = s_ref[...][0]
    dn = (((0,), (0,)), ((), ()))
    hnew_ref[...] = lax.dot_general(s, feat_ref[...][0], dn,
                                    preferred_element_type=jnp.float32)[None]
    adj0_ref[...] = lax.dot_general(s, as0_ref[...][0], dn,
                                    preferred_element_type=jnp.float32)[None]
    adj1_ref[...] = lax.dot_general(s, as1_ref[...][0], dn,
                                    preferred_element_type=jnp.float32)[None]


_AW = B * SHALF

_pool_call = pl.pallas_call(
    _pool_body,
    grid=(B,),
    in_specs=[
        pl.BlockSpec((1, NPG, SPAD), lambda g: (g, 0, 0)),
        pl.BlockSpec((1, NPG, DOUT), lambda g: (g, 0, 0)),
        pl.BlockSpec((1, NPG, _AW), lambda g: (g, 0, 0)),
        pl.BlockSpec((1, NPG, _AW), lambda g: (g, 0, 0)),
    ],
    out_specs=[
        pl.BlockSpec((1, SPAD, DOUT), lambda g: (g, 0, 0)),
        pl.BlockSpec((1, SPAD, _AW), lambda g: (g, 0, 0)),
        pl.BlockSpec((1, SPAD, _AW), lambda g: (g, 0, 0)),
    ],
    out_shape=[jax.ShapeDtypeStruct((B, SPAD, DOUT), jnp.float32),
               jax.ShapeDtypeStruct((B, SPAD, _AW), jnp.float32),
               jax.ShapeDtypeStruct((B, SPAD, _AW), jnp.float32)],
)


@jax.jit
def kernel(h, edge_index, W_feat, b_feat, W_pool, b_pool):
    src = edge_index[0]
    dst = edge_index[1]
    pad = E_PAD - E
    srcp = jnp.concatenate([src, jnp.zeros((pad,), jnp.int32)])
    dstp = jnp.concatenate([dst, jnp.full((pad,), N, jnp.int32)])
    rowidp = (dstp % NPR) * B + srcp // NPG
    rndp = dstp // NPR

    epack_a = jnp.stack([srcp.reshape(-1, CHUNK_A), dstp.reshape(-1, CHUNK_A)],
                        axis=1)
    epack_c = jnp.stack([srcp.reshape(-1, CHUNK), rowidp.reshape(-1, CHUNK),
                         rndp.reshape(-1, CHUNK)], axis=1)

    htab = h.reshape(2 * N, HALF)
    zero_a = jnp.zeros((ZROWS_A, HALF), jnp.float32)
    zero_d = jnp.zeros((ZROWS_A, 16), jnp.float32)
    ones_d = jnp.ones((CHUNK_A, 16), jnp.float32)
    summ_lo, summ_hi, degt = _seg_call()(htab, epack_a,
                                         zero_a, zero_d, ones_d)

    feat, s_pad = _dense_call(h, summ_lo[:N], summ_hi[:N], degt[:N],
                              W_feat, b_feat.reshape(1, DOUT),
                              jnp.pad(W_pool, ((0, 0), (0, SPAD - ASSIGN))),
                              jnp.pad(b_pool, (0, SPAD - ASSIGN)).reshape(1, SPAD))

    slo = s_pad[:, :SHALF]
    shi = s_pad[:, SHALF:]
    zero_c = jnp.zeros((ZROWS_C, SHALF), jnp.float32)
    as0, as1 = _as_call()(slo, shi, epack_c, zero_c)

    hnew, adj0, adj1 = _pool_call(s_pad.reshape(B, NPG, SPAD),
                                  feat.reshape(B, NPG, DOUT),
                                  as0.reshape(B, NPG, _AW),
                                  as1.reshape(B, NPG, _AW))

    h_new = hnew[:, :ASSIGN, :].reshape(B * ASSIGN, DOUT)
    a0 = adj0.reshape(B, SPAD, B, SHALF)
    a1 = adj1.reshape(B, SPAD, B, SHALF)
    adj = jnp.concatenate([a0, a1], axis=-1)[:, :ASSIGN, :, :ASSIGN]
    adj_new = adj.reshape(B * ASSIGN, B * ASSIGN)
    return (adj_new, h_new)
